# baseline (speedup 1.0000x reference)
import functools

import jax
import jax.numpy as jnp
from jax import lax
from jax.experimental import pallas as pl
from jax.experimental.pallas import tpu as pltpu
from jax.experimental.pallas import tpu_sc as plsc

_N = 4096
_D = 1024
_TOPK = 1024
_SC = 512
_RC = 128
_MC = 512
_GW = 32


def _topk_body(x_ref, p_ref, meta_ref, ycol_scr, rank_scr, nrm_scr):
    p_row = p_ref[...]
    nrm_scr[...] = jnp.sqrt(jnp.sum(p_row * p_row)).reshape(1, 1)
    nrm = nrm_scr[0, 0]
    p_col = p_row.reshape(_D, 1)

    def score_chunk(c, carry):
        base = pl.multiple_of(c * _SC, _SC)
        xc = x_ref[0, pl.ds(base, _SC), :]
        yc = None
        for q in range(4):
            dq = lax.dot_general(xc[:, q * 256:(q + 1) * 256],
                                 p_col[q * 256:(q + 1) * 256, :],
                                 (((1,), (0,)), ((), ())),
                                 preferred_element_type=jnp.float32)
            yc = dq if yc is None else yc + dq
        ycol_scr[pl.ds(base, _SC), :] = yc / nrm
        return carry

    lax.fori_loop(0, _N // _SC, score_chunk, 0)

    y_col = ycol_scr[...]
    n_i = lax.broadcasted_iota(jnp.int32, (_N, 128), 0)
    l_i = lax.broadcasted_iota(jnp.int32, (_N, 128), 1)
    z = jnp.where((n_i & 127) == l_i, 1.0, 0.0) * y_col
    s_i = lax.broadcasted_iota(jnp.int32, (32, _N), 0)
    n2_i = lax.broadcasted_iota(jnp.int32, (32, _N), 1)
    a32 = jnp.where((n2_i >> 7) == s_i, 1.0, 0.0)
    y32 = lax.dot_general(a32, z, (((1,), (0,)), ((), ())),
                          preferred_element_type=jnp.float32,
                          precision=jax.lax.Precision.HIGHEST)
    y3 = y32.reshape(1, 32, 128)
    col3 = (lax.broadcasted_iota(jnp.int32, (1, 32, 128), 1) * 128
            + lax.broadcasted_iota(jnp.int32, (1, 32, 128), 2))

    def rank_chunk(i, carry):
        base = pl.multiple_of(i * _RC, _RC)
        yr = ycol_scr[pl.ds(base, _RC), :].reshape(_RC, 1, 1)
        rid = lax.broadcasted_iota(jnp.int32, (_RC, 1, 1), 0) + i * _RC
        gt = y3 > yr
        tie = (y3 == yr) & (col3 < rid)
        cnt = jnp.sum(jnp.sum(jnp.where(gt | tie, 1.0, 0.0), axis=2), axis=1)
        rank_scr[pl.ds(base, _RC), :] = cnt.reshape(_RC, 1)
        return carry

    lax.fori_loop(0, _N // _RC, rank_chunk, 0)

    kio = lax.broadcasted_iota(jnp.int32, (1, _TOPK), 1).astype(jnp.float32)
    meta_ref[0] = jnp.zeros((_TOPK, 2), jnp.float32)

    def meta_chunk(c, carry):
        base = pl.multiple_of(c * _MC, _MC)
        rk = rank_scr[pl.ds(base, _MC), :]
        onehot = jnp.where(rk == kio, 1.0, 0.0)
        yv = ycol_scr[pl.ds(base, _MC), :]
        nv = (lax.broadcasted_iota(jnp.int32, (_MC, 1), 0) + c * _MC
              ).astype(jnp.float32)
        vals = jnp.concatenate([nv, jnp.tanh(yv)], axis=1)
        meta_ref[0] += lax.dot_general(onehot, vals, (((0,), (0,)), ((), ())),
                                       preferred_element_type=jnp.float32,
                                       precision=jax.lax.Precision.HIGHEST)
        return carry

    lax.fori_loop(0, _N // _MC, meta_chunk, 0)


def _sc_gather(table, idx):
    n_idx = idx.shape[0]
    info = plsc.get_sparse_core_info()
    nw = info.num_cores * info.num_subcores
    per_w = n_idx // nw
    mesh = plsc.VectorSubcoreMesh(core_axis_name="core",
                                  subcore_axis_name="subcore")

    @functools.partial(
        pl.kernel,
        out_type=jax.ShapeDtypeStruct((n_idx, _D), table.dtype),
        mesh=mesh,
        scratch_types=[
            pltpu.VMEM((_GW,), jnp.int32),
            pltpu.VMEM((_GW, _D), jnp.float32),
            pltpu.SemaphoreType.DMA,
        ],
    )
    def gather_kernel(x_hbm, i_hbm, o_hbm, idx_v, rows_v, sem):
        wid = (lax.axis_index("subcore") * info.num_cores
               + lax.axis_index("core"))

        @pl.loop(0, per_w // _GW)
        def _(c):
            base = wid * per_w + c * _GW
            pltpu.sync_copy(i_hbm.at[pl.ds(base, _GW)], idx_v)
            pltpu.async_copy(x_hbm.at[idx_v], rows_v, sem).wait()
            pltpu.sync_copy(rows_v, o_hbm.at[pl.ds(base, _GW)])

    return gather_kernel(table, idx)


def _gate_body(x_ref, g_ref, o_ref):
    o_ref[0] = x_ref[0] * g_ref[0]


def kernel(inputs, p):
    b = inputs.shape[0]
    p2 = p.reshape(1, _D)

    meta = pl.pallas_call(
        _topk_body,
        grid=(b,),
        in_specs=[
            pl.BlockSpec((1, _N, _D), lambda i: (i, 0, 0)),
            pl.BlockSpec((1, _D), lambda i: (0, 0)),
        ],
        out_specs=pl.BlockSpec((1, _TOPK, 2), lambda i: (i, 0, 0)),
        out_shape=jax.ShapeDtypeStruct((b, _TOPK, 2), jnp.float32),
        scratch_shapes=[
            pltpu.VMEM((_N, 1), jnp.float32),
            pltpu.VMEM((_N, 1), jnp.float32),
            pltpu.VMEM((1, 1), jnp.float32),
        ],
    )(inputs, p2)

    idx = (meta[:, :, 0].astype(jnp.int32)
           + (jnp.arange(b, dtype=jnp.int32) * _N)[:, None]).reshape(-1)
    gates = meta[:, :, 1:2]

    gathered = _sc_gather(inputs.reshape(b * _N, _D), idx)
    gathered = gathered.reshape(b, _TOPK, _D)

    return pl.pallas_call(
        _gate_body,
        grid=(b,),
        in_specs=[
            pl.BlockSpec((1, _TOPK, _D), lambda i: (i, 0, 0)),
            pl.BlockSpec((1, _TOPK, 1), lambda i: (i, 0, 0)),
        ],
        out_specs=pl.BlockSpec((1, _TOPK, _D), lambda i: (i, 0, 0)),
        out_shape=jax.ShapeDtypeStruct((b, _TOPK, _D), jnp.float32),
    )(gathered, gates)


# speedup vs baseline: 1.2873x; 1.2873x over previous
import functools

import jax
import jax.numpy as jnp
from jax import lax
from jax.experimental import pallas as pl
from jax.experimental.pallas import tpu as pltpu
from jax.experimental.pallas import tpu_sc as plsc

_N = 4096
_D = 1024
_TOPK = 1024
_SC = 512
_RC = 128
_MC = 512
_GW = 32


def _topk_body(x_ref, p_ref, meta_ref, ycol_scr, rank_scr, nrm_scr):
    p_row = p_ref[...]
    nrm_scr[...] = jnp.sqrt(jnp.sum(p_row * p_row)).reshape(1, 1)
    nrm = nrm_scr[0, 0]
    p_col = p_row.reshape(_D, 1)

    def score_chunk(c, carry):
        base = pl.multiple_of(c * _SC, _SC)
        xc = x_ref[0, pl.ds(base, _SC), :]
        yc = None
        for q in range(4):
            dq = lax.dot_general(xc[:, q * 256:(q + 1) * 256],
                                 p_col[q * 256:(q + 1) * 256, :],
                                 (((1,), (0,)), ((), ())),
                                 preferred_element_type=jnp.float32)
            yc = dq if yc is None else yc + dq
        ycol_scr[pl.ds(base, _SC), :] = yc / nrm
        return carry

    lax.fori_loop(0, _N // _SC, score_chunk, 0)

    y_col = ycol_scr[...]
    n_i = lax.broadcasted_iota(jnp.int32, (_N, 128), 0)
    l_i = lax.broadcasted_iota(jnp.int32, (_N, 128), 1)
    z = jnp.where((n_i & 127) == l_i, 1.0, 0.0) * y_col
    s_i = lax.broadcasted_iota(jnp.int32, (32, _N), 0)
    n2_i = lax.broadcasted_iota(jnp.int32, (32, _N), 1)
    a32 = jnp.where((n2_i >> 7) == s_i, 1.0, 0.0)
    y32 = lax.dot_general(a32, z, (((1,), (0,)), ((), ())),
                          preferred_element_type=jnp.float32,
                          precision=jax.lax.Precision.HIGHEST)
    y3 = y32.reshape(1, 32, 128)
    col3 = (lax.broadcasted_iota(jnp.int32, (1, 32, 128), 1) * 128
            + lax.broadcasted_iota(jnp.int32, (1, 32, 128), 2))

    def rank_chunk(i, carry):
        base = pl.multiple_of(i * _RC, _RC)
        yr = ycol_scr[pl.ds(base, _RC), :].reshape(_RC, 1, 1)
        rid = lax.broadcasted_iota(jnp.int32, (_RC, 1, 1), 0) + i * _RC
        gt = y3 > yr
        tie = (y3 == yr) & (col3 < rid)
        cnt = jnp.sum(jnp.sum(jnp.where(gt | tie, 1.0, 0.0), axis=2), axis=1)
        rank_scr[pl.ds(base, _RC), :] = cnt.reshape(_RC, 1)
        return carry

    lax.fori_loop(0, _N // _RC, rank_chunk, 0)

    ri = rank_scr[...].astype(jnp.int32)
    s8 = lax.broadcasted_iota(jnp.int32, (_N, 8), 1)
    hi = jnp.where((ri >> 7) == s8, 1.0, 0.0)
    l128 = lax.broadcasted_iota(jnp.int32, (_N, 128), 1)
    lo = jnp.where((ri & 127) == l128, 1.0, 0.0)
    nv = lax.broadcasted_iota(jnp.int32, (_N, 1), 0).astype(jnp.float32)
    zcat = jnp.concatenate([nv * lo, jnp.tanh(ycol_scr[...]) * lo], axis=1)
    meta_ref[0] = lax.dot_general(hi, zcat, (((0,), (0,)), ((), ())),
                                  preferred_element_type=jnp.float32,
                                  precision=jax.lax.Precision.HIGHEST)


def _sc_gather(table, idx):
    n_idx = idx.shape[0]
    info = plsc.get_sparse_core_info()
    nw = info.num_cores * info.num_subcores
    per_w = n_idx // nw
    mesh = plsc.VectorSubcoreMesh(core_axis_name="core",
                                  subcore_axis_name="subcore")

    @functools.partial(
        pl.kernel,
        out_type=jax.ShapeDtypeStruct((n_idx, _D), table.dtype),
        mesh=mesh,
        scratch_types=[
            pltpu.VMEM((_GW,), jnp.int32),
            pltpu.VMEM((_GW, _D), jnp.float32),
            pltpu.SemaphoreType.DMA,
        ],
    )
    def gather_kernel(x_hbm, i_hbm, o_hbm, idx_v, rows_v, sem):
        wid = (lax.axis_index("subcore") * info.num_cores
               + lax.axis_index("core"))

        @pl.loop(0, per_w // _GW)
        def _(c):
            base = wid * per_w + c * _GW
            pltpu.sync_copy(i_hbm.at[pl.ds(base, _GW)], idx_v)
            pltpu.async_copy(x_hbm.at[idx_v], rows_v, sem).wait()
            pltpu.sync_copy(rows_v, o_hbm.at[pl.ds(base, _GW)])

    return gather_kernel(table, idx)


def _gate_body(x_ref, g_ref, o_ref):
    o_ref[0] = x_ref[0] * g_ref[0]


def kernel(inputs, p):
    b = inputs.shape[0]
    p2 = p.reshape(1, _D)

    meta8 = pl.pallas_call(
        _topk_body,
        grid=(b,),
        in_specs=[
            pl.BlockSpec((1, _N, _D), lambda i: (i, 0, 0)),
            pl.BlockSpec((1, _D), lambda i: (0, 0)),
        ],
        out_specs=pl.BlockSpec((1, 8, 256), lambda i: (i, 0, 0)),
        out_shape=jax.ShapeDtypeStruct((b, 8, 256), jnp.float32),
        scratch_shapes=[
            pltpu.VMEM((_N, 1), jnp.float32),
            pltpu.VMEM((_N, 1), jnp.float32),
            pltpu.VMEM((1, 1), jnp.float32),
        ],
    )(inputs, p2)

    idx = (meta8[:, :, 0:128].reshape(b, _TOPK).astype(jnp.int32)
           + (jnp.arange(b, dtype=jnp.int32) * _N)[:, None]).reshape(-1)
    gates = meta8[:, :, 128:256].reshape(b, _TOPK, 1)

    gathered = _sc_gather(inputs.reshape(b * _N, _D), idx)
    gathered = gathered.reshape(b, _TOPK, _D)

    return pl.pallas_call(
        _gate_body,
        grid=(b,),
        in_specs=[
            pl.BlockSpec((1, _TOPK, _D), lambda i: (i, 0, 0)),
            pl.BlockSpec((1, _TOPK, 1), lambda i: (i, 0, 0)),
        ],
        out_specs=pl.BlockSpec((1, _TOPK, _D), lambda i: (i, 0, 0)),
        out_shape=jax.ShapeDtypeStruct((b, _TOPK, _D), jnp.float32),
    )(gathered, gates)


# speedup vs baseline: 1.3606x; 1.0569x over previous
import functools

import jax
import jax.numpy as jnp
from jax import lax
from jax.experimental import pallas as pl
from jax.experimental.pallas import tpu as pltpu
from jax.experimental.pallas import tpu_sc as plsc

_N = 4096
_D = 1024
_TOPK = 1024
_SC = 512
_RC = 256
_MC = 512
_GW = 32


def _topk_body(x_ref, p_ref, meta_ref, ycol_scr, rank_scr, nrm_scr):
    p_row = p_ref[...]
    nrm_scr[...] = jnp.sqrt(jnp.sum(p_row * p_row)).reshape(1, 1)
    nrm = nrm_scr[0, 0]
    p_col = p_row.reshape(_D, 1)

    def score_chunk(c, carry):
        base = pl.multiple_of(c * _SC, _SC)
        xc = x_ref[0, pl.ds(base, _SC), :]
        yc = None
        for q in range(4):
            dq = lax.dot_general(xc[:, q * 256:(q + 1) * 256],
                                 p_col[q * 256:(q + 1) * 256, :],
                                 (((1,), (0,)), ((), ())),
                                 preferred_element_type=jnp.float32)
            yc = dq if yc is None else yc + dq
        ycol_scr[pl.ds(base, _SC), :] = yc / nrm
        return carry

    lax.fori_loop(0, _N // _SC, score_chunk, 0)

    y_col = ycol_scr[...]
    n_i = lax.broadcasted_iota(jnp.int32, (_N, 128), 0)
    l_i = lax.broadcasted_iota(jnp.int32, (_N, 128), 1)
    z = jnp.where((n_i & 127) == l_i, 1.0, 0.0) * y_col
    s_i = lax.broadcasted_iota(jnp.int32, (32, _N), 0)
    n2_i = lax.broadcasted_iota(jnp.int32, (32, _N), 1)
    a32 = jnp.where((n2_i >> 7) == s_i, 1.0, 0.0)
    y32 = lax.dot_general(a32, z, (((1,), (0,)), ((), ())),
                          preferred_element_type=jnp.float32,
                          precision=jax.lax.Precision.HIGHEST)
    y3 = y32.reshape(1, 32, 128)
    col3 = (lax.broadcasted_iota(jnp.int32, (1, 32, 128), 1) * 128
            + lax.broadcasted_iota(jnp.int32, (1, 32, 128), 2))

    def rank_chunk(i, carry):
        base = pl.multiple_of(i * _RC, _RC)
        yr = ycol_scr[pl.ds(base, _RC), :].reshape(_RC, 1, 1)
        rid = lax.broadcasted_iota(jnp.int32, (_RC, 1, 1), 0) + i * _RC
        gt = y3 > yr
        tie = (y3 == yr) & (col3 < rid)
        cnt = jnp.sum(jnp.sum(jnp.where(gt | tie, 1.0, 0.0), axis=2), axis=1)
        rank_scr[pl.ds(base, _RC), :] = cnt.reshape(_RC, 1)
        return carry

    lax.fori_loop(0, _N // _RC, rank_chunk, 0)

    ri = rank_scr[...].astype(jnp.int32)
    s8 = lax.broadcasted_iota(jnp.int32, (_N, 8), 1)
    hi = jnp.where((ri >> 7) == s8, 1.0, 0.0)
    l128 = lax.broadcasted_iota(jnp.int32, (_N, 128), 1)
    lo = jnp.where((ri & 127) == l128, 1.0, 0.0)
    nv = lax.broadcasted_iota(jnp.int32, (_N, 1), 0).astype(jnp.float32)
    zcat = jnp.concatenate([nv * lo, jnp.tanh(ycol_scr[...]) * lo], axis=1)
    meta_ref[0] = lax.dot_general(hi, zcat, (((0,), (0,)), ((), ())),
                                  preferred_element_type=jnp.float32,
                                  precision=jax.lax.Precision.HIGHEST)


def _sc_gather(table, idx):
    n_idx = idx.shape[0]
    info = plsc.get_sparse_core_info()
    nw = info.num_cores * info.num_subcores
    per_w = n_idx // nw
    mesh = plsc.VectorSubcoreMesh(core_axis_name="core",
                                  subcore_axis_name="subcore")

    @functools.partial(
        pl.kernel,
        out_type=jax.ShapeDtypeStruct((n_idx, _D), table.dtype),
        mesh=mesh,
        scratch_types=[
            pltpu.VMEM((_GW,), jnp.int32),
            pltpu.VMEM((_GW, _D), jnp.float32),
            pltpu.SemaphoreType.DMA,
        ],
    )
    def gather_kernel(x_hbm, i_hbm, o_hbm, idx_v, rows_v, sem):
        wid = (lax.axis_index("subcore") * info.num_cores
               + lax.axis_index("core"))

        @pl.loop(0, per_w // _GW)
        def _(c):
            base = wid * per_w + c * _GW
            pltpu.sync_copy(i_hbm.at[pl.ds(base, _GW)], idx_v)
            pltpu.async_copy(x_hbm.at[idx_v], rows_v, sem).wait()
            pltpu.sync_copy(rows_v, o_hbm.at[pl.ds(base, _GW)])

    return gather_kernel(table, idx)


def _gate_body(x_ref, g_ref, o_ref):
    o_ref[0] = x_ref[0] * g_ref[0]


def kernel(inputs, p):
    b = inputs.shape[0]
    p2 = p.reshape(1, _D)

    meta8 = pl.pallas_call(
        _topk_body,
        grid=(b,),
        in_specs=[
            pl.BlockSpec((1, _N, _D), lambda i: (i, 0, 0)),
            pl.BlockSpec((1, _D), lambda i: (0, 0)),
        ],
        out_specs=pl.BlockSpec((1, 8, 256), lambda i: (i, 0, 0)),
        out_shape=jax.ShapeDtypeStruct((b, 8, 256), jnp.float32),
        scratch_shapes=[
            pltpu.VMEM((_N, 1), jnp.float32),
            pltpu.VMEM((_N, 1), jnp.float32),
            pltpu.VMEM((1, 1), jnp.float32),
        ],
    )(inputs, p2)

    idx = (meta8[:, :, 0:128].reshape(b, _TOPK).astype(jnp.int32)
           + (jnp.arange(b, dtype=jnp.int32) * _N)[:, None]).reshape(-1)
    gates = meta8[:, :, 128:256].reshape(b, _TOPK, 1)

    gathered = _sc_gather(inputs.reshape(b * _N, _D), idx)
    gathered = gathered.reshape(b, _TOPK, _D)

    return pl.pallas_call(
        _gate_body,
        grid=(b,),
        in_specs=[
            pl.BlockSpec((1, _TOPK, _D), lambda i: (i, 0, 0)),
            pl.BlockSpec((1, _TOPK, 1), lambda i: (i, 0, 0)),
        ],
        out_specs=pl.BlockSpec((1, _TOPK, _D), lambda i: (i, 0, 0)),
        out_shape=jax.ShapeDtypeStruct((b, _TOPK, _D), jnp.float32),
    )(gathered, gates)
